# baseline (speedup 1.0000x reference)
import functools
import math

import jax
import jax.numpy as jnp
from jax.experimental import pallas as pl
from jax.experimental.pallas import tpu as pltpu

N = 10000
E = 160000
H = 128

_BE = 1000
_BN = 1000


def _silu(v):
    return v * jax.nn.sigmoid(v)


def _edge_body(hr, hc, xr, xc, w1h1, w1h2, w1d, b1, w2, b2, aw, ab,
               cw1, cb1, cw2, msg, xmsg):
    rel = xr[...] - xc[...]
    dist = jnp.sqrt(jnp.sum(rel * rel, axis=-1, keepdims=True))
    t1 = (jnp.dot(hr[...], w1h1[...], preferred_element_type=jnp.float32)
          + jnp.dot(hc[...], w1h2[...], preferred_element_type=jnp.float32)
          + dist * w1d[...] + b1[...])
    t1 = _silu(t1)
    m = jnp.dot(t1, w2[...], preferred_element_type=jnp.float32) + b2[...]
    m = _silu(m)
    att = jax.nn.sigmoid(jnp.sum(m * aw[...], axis=-1, keepdims=True)
                         + ab[...])
    msg[...] = att * m
    c1 = _silu(jnp.dot(m, cw1[...], preferred_element_type=jnp.float32)
               + cb1[...])
    cw = jnp.sum(c1 * cw2[...], axis=-1, keepdims=True)
    xmsg[...] = cw * rel / (dist + 1e-8)


def _edge_block_call(hr, hc, xr, xc, wts):
    n_e = hr.shape[0]
    grid = n_e // _BE
    full = lambda s: pl.BlockSpec(s, lambda i: (0,) * len(s))
    eb = lambda d: pl.BlockSpec((_BE, d), lambda i: (i, 0))
    return pl.pallas_call(
        _edge_body,
        grid=(grid,),
        in_specs=[eb(H), eb(H), eb(8), eb(8),
                  full((H, H)), full((H, H)), full((1, H)), full((1, H)),
                  full((H, H)), full((1, H)), full((1, H)), full((1, 1)),
                  full((H, H)), full((1, H)), full((1, H))],
        out_specs=[eb(H), eb(8)],
        out_shape=[jax.ShapeDtypeStruct((n_e, H), jnp.float32),
                   jax.ShapeDtypeStruct((n_e, 8), jnp.float32)],
    )(hr, hc, xr, xc, *wts)


def _node_body(ln, h, mi, w1a, w1b, b1, w2, b2, g, bv, out):
    t = (jnp.dot(h[...], w1a[...], preferred_element_type=jnp.float32)
         + jnp.dot(mi[...], w1b[...], preferred_element_type=jnp.float32)
         + b1[...])
    t = _silu(t)
    hn = jnp.dot(t, w2[...], preferred_element_type=jnp.float32) + b2[...]
    hnew = h[...] + hn
    if ln:
        mu = jnp.mean(hnew, axis=-1, keepdims=True)
        va = jnp.mean((hnew - mu) ** 2, axis=-1, keepdims=True)
        hnew = (hnew - mu) / jnp.sqrt(va + 1e-5) * g[...] + bv[...]
    out[...] = hnew


def _node_call(ln, h, mi, w1a, w1b, b1, w2, b2, g, bv):
    grid = N // _BN
    full = lambda s: pl.BlockSpec(s, lambda i: (0,) * len(s))
    nb = pl.BlockSpec((_BN, H), lambda i: (i, 0))
    return pl.pallas_call(
        functools.partial(_node_body, ln),
        grid=(grid,),
        in_specs=[nb, nb, full((H, H)), full((H, H)), full((1, H)),
                  full((H, H)), full((1, H)), full((1, H)), full((1, H))],
        out_specs=nb,
        out_shape=jax.ShapeDtypeStruct((N, H), jnp.float32),
    )(h, mi, w1a, w1b, b1, w2, b2, g, bv)


def _precompute(t, af, dc, cc, p):
    silu = jax.nn.silu
    half = H // 2
    freqs = jnp.exp(jnp.arange(half, dtype=jnp.float32)
                    * (-(math.log(10000.0) / (half - 1))))
    te = t.astype(jnp.float32)[:, None] * freqs[None, :]
    temb = jnp.concatenate([jnp.sin(te), jnp.cos(te)], axis=-1)
    a = silu(af @ p['ce_aW1'] + p['ce_ab1']) @ p['ce_aW2'] + p['ce_ab2']
    a_emb = a.mean(axis=0, keepdims=True)
    d = silu(dc @ p['ce_dW1'] + p['ce_db1']) @ p['ce_dW2'] + p['ce_db2']
    d_emb = d.mean(axis=0, keepdims=True)
    c = silu(cc @ p['ce_cW1'] + p['ce_cb1']) @ p['ce_cW2'] + p['ce_cb2']
    c_emb = c.mean(axis=0, keepdims=True)
    comb = jnp.concatenate([a_emb, d_emb, c_emb], axis=-1)
    z = comb @ p['ce_fW1'] + p['ce_fb1']
    mu = z.mean(axis=-1, keepdims=True)
    va = ((z - mu) ** 2).mean(axis=-1, keepdims=True)
    z = (z - mu) / jnp.sqrt(va + 1e-5) * p['ce_fg'] + p['ce_fbe']
    cond = silu(z) @ p['ce_fW2'] + p['ce_fb2']
    tproj = temb @ p['tpW'] + p['tpb']
    cproj = cond @ p['cpW'] + p['cpb']
    return tproj + cproj


def kernel(h, x, edge_index, t, anchor_features, distance_constraints,
           coordination_constraints, params):
    p = params
    row = edge_index[0]
    col = edge_index[1]
    h = h + _precompute(t, anchor_features, distance_constraints,
                        coordination_constraints, p)
    x8 = jnp.pad(x, ((0, 0), (0, 5)))
    for i in range(2):
        hr = jnp.take(h, row, axis=0)
        hc = jnp.take(h, col, axis=0)
        xr = jnp.take(x8, row, axis=0)
        xc = jnp.take(x8, col, axis=0)
        wts = (p['eW1'][i, :H], p['eW1'][i, H:2 * H], p['eW1'][i, 2 * H:],
               p['eb1'][i][None], p['eW2'][i], p['eb2'][i][None],
               p['aW'][i].T, p['ab'][i][None],
               p['cW1'][i], p['cb1'][i][None], p['cW2'][i].T)
        msg, xmsg = _edge_block_call(hr, hc, xr, xc, wts)
        mi = jax.ops.segment_sum(msg, row, num_segments=N)
        xacc = jax.ops.segment_sum(xmsg, row, num_segments=N)
        h = _node_call(i == 1, h, mi,
                       p['nW1'][i, :H], p['nW1'][i, H:], p['nb1'][i][None],
                       p['nW2'][i], p['nb2'][i][None],
                       p['ln_g'][None], p['ln_b'][None])
        x8 = x8 + xacc
    return h, x8[:, :3]


# speedup vs baseline: 3.4575x; 3.4575x over previous
import dataclasses
import functools
import math

import jax
import jax.numpy as jnp
from jax import lax
from jax.experimental import pallas as pl
from jax.experimental.pallas import tpu as pltpu
from jax.experimental.pallas import tpu_sc as plsc

N = 10000
E = 160000
H = 128

_NC = 2
_NS = 16
_NW = _NC * _NS
_C = 128
_EROWS = 1280
_EPAD = _EROWS * _C
_RPW = _EROWS // _NW
_NP = 10240

_BE = 1024
_BN = 1000

_sc_mesh = plsc.VectorSubcoreMesh(core_axis_name="c", subcore_axis_name="s")

_sc_cp = pltpu.CompilerParams()
if "needs_layout_passes" in pltpu.CompilerParams.__dataclass_fields__:
    _sc_cp = dataclasses.replace(_sc_cp, needs_layout_passes=False)


def _sc_gather(h2b, x4, rowg, colg):
    @functools.partial(
        pl.kernel, mesh=_sc_mesh,
        out_type=[jax.ShapeDtypeStruct((_EPAD, H), jnp.float32),
                  jax.ShapeDtypeStruct((_EPAD, H), jnp.float32),
                  jax.ShapeDtypeStruct((8, _EPAD), jnp.float32)],
        scratch_types=[pltpu.VMEM((_RPW, _C), jnp.int32),
                       pltpu.VMEM((_RPW, _C), jnp.int32),
                       pltpu.VMEM((_C, H), jnp.float32),
                       pltpu.VMEM((_C, H), jnp.float32),
                       pltpu.VMEM((4 * N,), jnp.float32),
                       pltpu.VMEM((8, _C), jnp.float32),
                       pltpu.SemaphoreType.DMA],
        compiler_params=_sc_cp,
    )
    def k(h_hbm, x_hbm, ri_hbm, ci_hbm, hr_hbm, hc_hbm, rel_hbm,
          ribuf, cibuf, hbuf, cbuf, x4v, relbuf, sem):
        wid = lax.axis_index("s") * _NC + lax.axis_index("c")
        base = wid * _RPW
        pltpu.sync_copy(ri_hbm.at[pl.ds(base, _RPW)], ribuf)
        pltpu.sync_copy(ci_hbm.at[pl.ds(base, _RPW)], cibuf)
        pltpu.sync_copy(x_hbm, x4v)
        zero16 = jnp.zeros((16,), jnp.float32)
        for r in range(4, 8):
            for kk in range(8):
                relbuf[r, pl.ds(kk * 16, 16)] = zero16

        @pl.loop(0, _RPW)
        def _(j):
            dh = pltpu.async_copy(h_hbm.at[ribuf.at[j]], hbuf, sem)
            dc = pltpu.async_copy(h_hbm.at[cibuf.at[j]], cbuf, sem)
            for kk in range(8):
                sl = pl.ds(kk * 16, 16)
                ir = ribuf[j, sl] * 4
                ic = cibuf[j, sl] * 4
                d2 = zero16
                for comp in range(3):
                    cidx = jnp.full((16,), comp, jnp.int32)
                    vr = plsc.load_gather(x4v, [ir + cidx])
                    vc = plsc.load_gather(x4v, [ic + cidx])
                    rr = vr - vc
                    relbuf[comp, sl] = rr
                    d2 = d2 + rr * rr
                relbuf[3, sl] = d2
            e0 = (base + j) * _C
            dh.wait()
            dc.wait()
            pltpu.sync_copy(hbuf, hr_hbm.at[pl.ds(e0, _C)])
            pltpu.sync_copy(cbuf, hc_hbm.at[pl.ds(e0, _C)])
            pltpu.sync_copy(relbuf, rel_hbm.at[:, pl.ds(e0, _C)])

    return k(h2b, x4, rowg, colg)


def _sc_scatter(msg, xmsg, rows, zm):
    rps = _NP // _NS
    rps_e = _EROWS // _NS

    @functools.partial(
        pl.kernel, mesh=_sc_mesh,
        out_type=[jax.ShapeDtypeStruct((_NP, H), jnp.float32),
                  jax.ShapeDtypeStruct((_NP, H), jnp.float32)],
        scratch_types=[pltpu.VMEM((rps_e, _C), jnp.int32),
                       pltpu.VMEM((_C, H), jnp.float32),
                       pltpu.VMEM_SHARED((_NP, H), jnp.float32),
                       pltpu.SemaphoreType.DMA],
        compiler_params=_sc_cp,
    )
    def k(m_hbm, xm_hbm, ri_hbm, zm_hbm, mo_hbm, xo_hbm,
          ibuf, mbuf, acc, sem):
        c = lax.axis_index("c")
        s = lax.axis_index("s")
        pltpu.sync_copy(zm_hbm, acc.at[pl.ds(s * rps, rps)])
        plsc.subcore_barrier()
        base = s * rps_e
        pltpu.sync_copy(ri_hbm.at[pl.ds(base, rps_e)], ibuf)

        @pl.loop(0, rps_e)
        def _(j):
            e0 = (base + j) * _C

            @pl.when(c == 0)
            def _():
                pltpu.async_copy(m_hbm.at[pl.ds(e0, _C)], mbuf, sem).wait()

            @pl.when(c == 1)
            def _():
                pltpu.async_copy(xm_hbm.at[pl.ds(e0, _C)], mbuf, sem).wait()

            pltpu.sync_copy(mbuf, acc.at[ibuf.at[j]], add=True)

        plsc.subcore_barrier()

        @pl.when(c == 0)
        def _():
            pltpu.sync_copy(acc.at[pl.ds(s * rps, rps)],
                            mo_hbm.at[pl.ds(s * rps, rps)])

        @pl.when(c == 1)
        def _():
            pltpu.sync_copy(acc.at[pl.ds(s * rps, rps)],
                            xo_hbm.at[pl.ds(s * rps, rps)])

    return k(msg, xmsg, rows, zm)


def _silu(v):
    return v * jax.nn.sigmoid(v)


def _edge_body(hr, hc, rel8, eye8, msk8, w1h1, w1h2, w1d, b1, w2, b2,
               aw, ab, cw1, cb1, cw2, msg, xout):
    r8 = lax.dot_general(rel8[...], eye8[...], (((0,), (0,)), ((), ())),
                         preferred_element_type=jnp.float32)
    dist = jnp.sqrt(r8[:, 3:4])
    hrb = hr[...].astype(jnp.bfloat16)
    hcb = hc[...].astype(jnp.bfloat16)
    t1 = (jnp.dot(hrb, w1h1[...], preferred_element_type=jnp.float32)
          + jnp.dot(hcb, w1h2[...], preferred_element_type=jnp.float32)
          + dist * w1d[...] + b1[...])
    t1 = _silu(t1).astype(jnp.bfloat16)
    m = jnp.dot(t1, w2[...], preferred_element_type=jnp.float32) + b2[...]
    m = _silu(m)
    att = jax.nn.sigmoid(jnp.sum(m * aw[...], axis=-1, keepdims=True)
                         + ab[...])
    msg[...] = att * m
    mb = m.astype(jnp.bfloat16)
    c1 = _silu(jnp.dot(mb, cw1[...], preferred_element_type=jnp.float32)
               + cb1[...])
    cw = jnp.sum(c1 * cw2[...], axis=-1, keepdims=True)
    xm = (cw * r8 / (dist + 1e-8)) * msk8[...]
    xout[...] = jnp.concatenate(
        [xm, jnp.zeros((xm.shape[0], H - 8), jnp.float32)], axis=1)


def _edge_block_call(hr, hc, rel8, eye8, msk8, wts):
    grid = _EPAD // _BE
    full = lambda s: pl.BlockSpec(s, lambda i: (0,) * len(s))
    eb = lambda d: pl.BlockSpec((_BE, d), lambda i: (i, 0))
    return pl.pallas_call(
        _edge_body,
        grid=(grid,),
        in_specs=[eb(H), eb(H), pl.BlockSpec((8, _BE), lambda i: (0, i)),
                  full((8, 8)), full((1, 8)),
                  full((H, H)), full((H, H)), full((1, H)), full((1, H)),
                  full((H, H)), full((1, H)), full((1, H)), full((1, 1)),
                  full((H, H)), full((1, H)), full((1, H))],
        out_specs=[eb(H), eb(H)],
        out_shape=[jax.ShapeDtypeStruct((_EPAD, H), jnp.float32),
                   jax.ShapeDtypeStruct((_EPAD, H), jnp.float32)],
    )(hr, hc, rel8, eye8, msk8, *wts)


def _node_body(ln, h, mi, w1a, w1b, b1, w2, b2, g, bv, out):
    t = (jnp.dot(h[...], w1a[...], preferred_element_type=jnp.float32)
         + jnp.dot(mi[...], w1b[...], preferred_element_type=jnp.float32)
         + b1[...])
    t = _silu(t)
    hn = jnp.dot(t, w2[...], preferred_element_type=jnp.float32) + b2[...]
    hnew = h[...] + hn
    if ln:
        mu = jnp.mean(hnew, axis=-1, keepdims=True)
        va = jnp.mean((hnew - mu) ** 2, axis=-1, keepdims=True)
        hnew = (hnew - mu) / jnp.sqrt(va + 1e-5) * g[...] + bv[...]
    out[...] = hnew


def _node_call(ln, h, mi, w1a, w1b, b1, w2, b2, g, bv):
    grid = N // _BN
    full = lambda s: pl.BlockSpec(s, lambda i: (0,) * len(s))
    nb = pl.BlockSpec((_BN, H), lambda i: (i, 0))
    return pl.pallas_call(
        functools.partial(_node_body, ln),
        grid=(grid,),
        in_specs=[nb, nb, full((H, H)), full((H, H)), full((1, H)),
                  full((H, H)), full((1, H)), full((1, H)), full((1, H))],
        out_specs=nb,
        out_shape=jax.ShapeDtypeStruct((N, H), jnp.float32),
    )(h, mi, w1a, w1b, b1, w2, b2, g, bv)


def _precompute(t, af, dc, cc, p):
    silu = jax.nn.silu
    half = H // 2
    freqs = jnp.exp(jnp.arange(half, dtype=jnp.float32)
                    * (-(math.log(10000.0) / (half - 1))))
    te = t.astype(jnp.float32)[:, None] * freqs[None, :]
    temb = jnp.concatenate([jnp.sin(te), jnp.cos(te)], axis=-1)
    a = silu(af @ p['ce_aW1'] + p['ce_ab1']) @ p['ce_aW2'] + p['ce_ab2']
    a_emb = a.mean(axis=0, keepdims=True)
    d = silu(dc @ p['ce_dW1'] + p['ce_db1']) @ p['ce_dW2'] + p['ce_db2']
    d_emb = d.mean(axis=0, keepdims=True)
    c = silu(cc @ p['ce_cW1'] + p['ce_cb1']) @ p['ce_cW2'] + p['ce_cb2']
    c_emb = c.mean(axis=0, keepdims=True)
    comb = jnp.concatenate([a_emb, d_emb, c_emb], axis=-1)
    z = comb @ p['ce_fW1'] + p['ce_fb1']
    mu = z.mean(axis=-1, keepdims=True)
    va = ((z - mu) ** 2).mean(axis=-1, keepdims=True)
    z = (z - mu) / jnp.sqrt(va + 1e-5) * p['ce_fg'] + p['ce_fbe']
    cond = silu(z) @ p['ce_fW2'] + p['ce_fb2']
    tproj = temb @ p['tpW'] + p['tpb']
    cproj = cond @ p['cpW'] + p['cpb']
    return tproj + cproj


def kernel(h, x, edge_index, t, anchor_features, distance_constraints,
           coordination_constraints, params):
    p = params
    npad = _EPAD - E
    gpad = (jnp.arange(npad, dtype=jnp.int32) * 37) % N
    rowg = jnp.concatenate([edge_index[0], gpad]).reshape(_EROWS, _C)
    colg = jnp.concatenate([edge_index[1], gpad]).reshape(_EROWS, _C)
    spad = N + (jnp.arange(npad, dtype=jnp.int32) % (_NP - N))
    rows = jnp.concatenate([edge_index[0], spad]).reshape(_EROWS, _C)
    zm = jnp.zeros((_NP // _NS, H), jnp.float32)
    eye8 = jnp.eye(8, dtype=jnp.float32)
    msk8 = jnp.array([[1., 1., 1., 0., 0., 0., 0., 0.]], jnp.float32)

    h = h + _precompute(t, anchor_features, distance_constraints,
                        coordination_constraints, p)
    for i in range(2):
        x4 = jnp.pad(x, ((0, 0), (0, 1))).reshape(-1)
        hr, hc, rel8 = _sc_gather(h, x4, rowg, colg)
        bf = jnp.bfloat16
        wts = (p['eW1'][i, :H].astype(bf), p['eW1'][i, H:2 * H].astype(bf),
               p['eW1'][i, 2 * H:], p['eb1'][i][None],
               p['eW2'][i].astype(bf), p['eb2'][i][None],
               p['aW'][i].T, p['ab'][i][None],
               p['cW1'][i].astype(bf), p['cb1'][i][None], p['cW2'][i].T)
        msg, xmsg = _edge_block_call(hr, hc, rel8, eye8, msk8, wts)
        mo, xo = _sc_scatter(msg, xmsg, rows, zm)
        h = _node_call(i == 1, h, mo[:N],
                       p['nW1'][i, :H], p['nW1'][i, H:], p['nb1'][i][None],
                       p['nW2'][i], p['nb2'][i][None],
                       p['ln_g'][None], p['ln_b'][None])
        x = x + xo[:N, :3]
    return h, x


# speedup vs baseline: 4.1742x; 1.2073x over previous
import dataclasses
import functools
import math

import jax
import jax.numpy as jnp
from jax import lax
from jax.experimental import pallas as pl
from jax.experimental.pallas import tpu as pltpu
from jax.experimental.pallas import tpu_sc as plsc

N = 10000
E = 160000
H = 128

_NC = 2
_NS = 16
_NW = _NC * _NS
_C = 128
_EROWS = 1280
_EPAD = _EROWS * _C
_RPW = _EROWS // _NW
_NP = 10240

_BE = 1024
_BN = 1000

_sc_mesh = plsc.VectorSubcoreMesh(core_axis_name="c", subcore_axis_name="s")

_sc_cp = pltpu.CompilerParams()
if "needs_layout_passes" in pltpu.CompilerParams.__dataclass_fields__:
    _sc_cp = dataclasses.replace(_sc_cp, needs_layout_passes=False)


def _sc_gather(h2b, x4, rowg, colg):
    @functools.partial(
        pl.kernel, mesh=_sc_mesh,
        out_type=[jax.ShapeDtypeStruct((_EPAD, H), jnp.float32),
                  jax.ShapeDtypeStruct((_EPAD, H), jnp.float32),
                  jax.ShapeDtypeStruct((8, _EPAD), jnp.float32)],
        scratch_types=[pltpu.VMEM((_RPW, _C), jnp.int32),
                       pltpu.VMEM((_RPW, _C), jnp.int32),
                       pltpu.VMEM((_C, H), jnp.float32),
                       pltpu.VMEM((_C, H), jnp.float32),
                       pltpu.VMEM((4 * N,), jnp.float32),
                       pltpu.VMEM((8, _C), jnp.float32),
                       pltpu.SemaphoreType.DMA],
        compiler_params=_sc_cp,
    )
    def k(h_hbm, x_hbm, ri_hbm, ci_hbm, hr_hbm, hc_hbm, rel_hbm,
          ribuf, cibuf, hbuf, cbuf, x4v, relbuf, sem):
        wid = lax.axis_index("s") * _NC + lax.axis_index("c")
        base = wid * _RPW
        pltpu.sync_copy(ri_hbm.at[pl.ds(base, _RPW)], ribuf)
        pltpu.sync_copy(ci_hbm.at[pl.ds(base, _RPW)], cibuf)
        pltpu.sync_copy(x_hbm, x4v)
        zero16 = jnp.zeros((16,), jnp.float32)
        for r in range(4, 8):
            for kk in range(8):
                relbuf[r, pl.ds(kk * 16, 16)] = zero16

        @pl.loop(0, _RPW)
        def _(j):
            dh = pltpu.async_copy(h_hbm.at[ribuf.at[j]], hbuf, sem)
            dc = pltpu.async_copy(h_hbm.at[cibuf.at[j]], cbuf, sem)
            for kk in range(8):
                sl = pl.ds(kk * 16, 16)
                ir = ribuf[j, sl] * 4
                ic = cibuf[j, sl] * 4
                d2 = zero16
                for comp in range(3):
                    cidx = jnp.full((16,), comp, jnp.int32)
                    vr = plsc.load_gather(x4v, [ir + cidx])
                    vc = plsc.load_gather(x4v, [ic + cidx])
                    rr = vr - vc
                    relbuf[comp, sl] = rr
                    d2 = d2 + rr * rr
                relbuf[3, sl] = d2
            e0 = (base + j) * _C
            dh.wait()
            dc.wait()
            pltpu.sync_copy(hbuf, hr_hbm.at[pl.ds(e0, _C)])
            pltpu.sync_copy(cbuf, hc_hbm.at[pl.ds(e0, _C)])
            pltpu.sync_copy(relbuf, rel_hbm.at[:, pl.ds(e0, _C)])

    return k(h2b, x4, rowg, colg)


def _sc_scatter(msg, xmsg, rows, zm):
    rps = _NP // _NS
    rps_e = _EROWS // _NS

    @functools.partial(
        pl.kernel, mesh=_sc_mesh,
        out_type=[jax.ShapeDtypeStruct((_NP, H), jnp.float32),
                  jax.ShapeDtypeStruct((_NP, H), jnp.float32)],
        scratch_types=[pltpu.VMEM((rps_e, _C), jnp.int32),
                       pltpu.VMEM((_C, H), jnp.float32),
                       pltpu.VMEM_SHARED((_NP, H), jnp.float32),
                       pltpu.SemaphoreType.DMA],
        compiler_params=_sc_cp,
    )
    def k(m_hbm, xm_hbm, ri_hbm, zm_hbm, mo_hbm, xo_hbm,
          ibuf, mbuf, acc, sem):
        c = lax.axis_index("c")
        s = lax.axis_index("s")
        pltpu.sync_copy(zm_hbm, acc.at[pl.ds(s * rps, rps)])
        plsc.subcore_barrier()
        base = s * rps_e
        pltpu.sync_copy(ri_hbm.at[pl.ds(base, rps_e)], ibuf)

        @pl.loop(0, rps_e)
        def _(j):
            e0 = (base + j) * _C

            @pl.when(c == 0)
            def _():
                pltpu.async_copy(m_hbm.at[pl.ds(e0, _C)], mbuf, sem).wait()

            @pl.when(c == 1)
            def _():
                pltpu.async_copy(xm_hbm.at[pl.ds(e0, _C)], mbuf, sem).wait()

            pltpu.sync_copy(mbuf, acc.at[ibuf.at[j]], add=True)

        plsc.subcore_barrier()

        @pl.when(c == 0)
        def _():
            pltpu.sync_copy(acc.at[pl.ds(s * rps, rps)],
                            mo_hbm.at[pl.ds(s * rps, rps)])

        @pl.when(c == 1)
        def _():
            pltpu.sync_copy(acc.at[pl.ds(s * rps, rps)],
                            xo_hbm.at[pl.ds(s * rps, rps)])

    return k(msg, xmsg, rows, zm)


def _silu(v):
    return v * jax.nn.sigmoid(v)


def _edge_body(hr, hc, rel8, eye8, msk8, w1h1, w1h2, w1d, b1, w2, b2,
               aw, ab, cw1, cb1, cw2, msg, xout):
    bf = jnp.bfloat16
    f32 = jnp.float32
    r8 = lax.dot_general(rel8[...], eye8[...], (((0,), (0,)), ((), ())),
                         preferred_element_type=f32)
    dist = jnp.sqrt(r8[:, 3:4])
    t1 = (jnp.dot(hr[...].astype(bf), w1h1[...],
                  preferred_element_type=f32)
          + jnp.dot(hc[...].astype(bf), w1h2[...],
                    preferred_element_type=f32)
          + dist * w1d[...] + b1[...]).astype(bf)
    t1 = _silu(t1)
    m = (jnp.dot(t1, w2[...], preferred_element_type=f32)
         + b2[...]).astype(bf)
    m = _silu(m)
    att = jax.nn.sigmoid(jnp.dot(m, aw[...], preferred_element_type=f32)
                         + ab[...])
    msg[...] = att * m.astype(f32)
    c1 = _silu((jnp.dot(m, cw1[...], preferred_element_type=f32)
                + cb1[...]).astype(bf))
    cw = jnp.dot(c1, cw2[...], preferred_element_type=f32)
    xm = (cw * r8 / (dist + 1e-8)) * msk8[...]
    xout[...] = jnp.concatenate(
        [xm, jnp.zeros((xm.shape[0], H - 8), f32)], axis=1)


def _edge_block_call(hr, hc, rel8, eye8, msk8, wts):
    grid = _EPAD // _BE
    full = lambda s: pl.BlockSpec(s, lambda i: (0,) * len(s))
    eb = lambda d: pl.BlockSpec((_BE, d), lambda i: (i, 0))
    return pl.pallas_call(
        _edge_body,
        grid=(grid,),
        in_specs=[eb(H), eb(H), pl.BlockSpec((8, _BE), lambda i: (0, i)),
                  full((8, 8)), full((1, 8)),
                  full((H, H)), full((H, H)), full((1, H)), full((1, H)),
                  full((H, H)), full((1, H)), full((H, 1)), full((1, 1)),
                  full((H, H)), full((1, H)), full((H, 1))],
        out_specs=[eb(H), eb(H)],
        out_shape=[jax.ShapeDtypeStruct((_EPAD, H), jnp.float32),
                   jax.ShapeDtypeStruct((_EPAD, H), jnp.float32)],
        compiler_params=pltpu.CompilerParams(
            dimension_semantics=("parallel",)),
    )(hr, hc, rel8, eye8, msk8, *wts)


def _node_body(ln, h, mi, w1a, w1b, b1, w2, b2, g, bv, out):
    t = (jnp.dot(h[...], w1a[...], preferred_element_type=jnp.float32)
         + jnp.dot(mi[...], w1b[...], preferred_element_type=jnp.float32)
         + b1[...])
    t = _silu(t)
    hn = jnp.dot(t, w2[...], preferred_element_type=jnp.float32) + b2[...]
    hnew = h[...] + hn
    if ln:
        mu = jnp.mean(hnew, axis=-1, keepdims=True)
        va = jnp.mean((hnew - mu) ** 2, axis=-1, keepdims=True)
        hnew = (hnew - mu) / jnp.sqrt(va + 1e-5) * g[...] + bv[...]
    out[...] = hnew


def _node_call(ln, h, mi, w1a, w1b, b1, w2, b2, g, bv):
    grid = N // _BN
    full = lambda s: pl.BlockSpec(s, lambda i: (0,) * len(s))
    nb = pl.BlockSpec((_BN, H), lambda i: (i, 0))
    return pl.pallas_call(
        functools.partial(_node_body, ln),
        grid=(grid,),
        in_specs=[nb, nb, full((H, H)), full((H, H)), full((1, H)),
                  full((H, H)), full((1, H)), full((1, H)), full((1, H))],
        out_specs=nb,
        out_shape=jax.ShapeDtypeStruct((N, H), jnp.float32),
        compiler_params=pltpu.CompilerParams(
            dimension_semantics=("parallel",)),
    )(h, mi, w1a, w1b, b1, w2, b2, g, bv)


def _precompute(t, af, dc, cc, p):
    silu = jax.nn.silu
    half = H // 2
    freqs = jnp.exp(jnp.arange(half, dtype=jnp.float32)
                    * (-(math.log(10000.0) / (half - 1))))
    te = t.astype(jnp.float32)[:, None] * freqs[None, :]
    temb = jnp.concatenate([jnp.sin(te), jnp.cos(te)], axis=-1)
    a = silu(af @ p['ce_aW1'] + p['ce_ab1']) @ p['ce_aW2'] + p['ce_ab2']
    a_emb = a.mean(axis=0, keepdims=True)
    d = silu(dc @ p['ce_dW1'] + p['ce_db1']) @ p['ce_dW2'] + p['ce_db2']
    d_emb = d.mean(axis=0, keepdims=True)
    c = silu(cc @ p['ce_cW1'] + p['ce_cb1']) @ p['ce_cW2'] + p['ce_cb2']
    c_emb = c.mean(axis=0, keepdims=True)
    comb = jnp.concatenate([a_emb, d_emb, c_emb], axis=-1)
    z = comb @ p['ce_fW1'] + p['ce_fb1']
    mu = z.mean(axis=-1, keepdims=True)
    va = ((z - mu) ** 2).mean(axis=-1, keepdims=True)
    z = (z - mu) / jnp.sqrt(va + 1e-5) * p['ce_fg'] + p['ce_fbe']
    cond = silu(z) @ p['ce_fW2'] + p['ce_fb2']
    tproj = temb @ p['tpW'] + p['tpb']
    cproj = cond @ p['cpW'] + p['cpb']
    return tproj + cproj


def kernel(h, x, edge_index, t, anchor_features, distance_constraints,
           coordination_constraints, params):
    p = params
    npad = _EPAD - E
    gpad = (jnp.arange(npad, dtype=jnp.int32) * 37) % N
    rowg = jnp.concatenate([edge_index[0], gpad]).reshape(_EROWS, _C)
    colg = jnp.concatenate([edge_index[1], gpad]).reshape(_EROWS, _C)
    spad = N + (jnp.arange(npad, dtype=jnp.int32) % (_NP - N))
    rows = jnp.concatenate([edge_index[0], spad]).reshape(_EROWS, _C)
    zm = jnp.zeros((_NP // _NS, H), jnp.float32)
    eye8 = jnp.eye(8, dtype=jnp.float32)
    msk8 = jnp.array([[1., 1., 1., 0., 0., 0., 0., 0.]], jnp.float32)

    h = h + _precompute(t, anchor_features, distance_constraints,
                        coordination_constraints, p)
    for i in range(2):
        x4 = jnp.pad(x, ((0, 0), (0, 1))).reshape(-1)
        hr, hc, rel8 = _sc_gather(h, x4, rowg, colg)
        bf = jnp.bfloat16
        wts = (p['eW1'][i, :H].astype(bf), p['eW1'][i, H:2 * H].astype(bf),
               p['eW1'][i, 2 * H:], p['eb1'][i][None],
               p['eW2'][i].astype(bf), p['eb2'][i][None],
               p['aW'][i].astype(bf), p['ab'][i][None],
               p['cW1'][i].astype(bf), p['cb1'][i][None],
               p['cW2'][i].astype(bf))
        msg, xmsg = _edge_block_call(hr, hc, rel8, eye8, msk8, wts)
        mo, xo = _sc_scatter(msg, xmsg, rows, zm)
        h = _node_call(i == 1, h, mo[:N],
                       p['nW1'][i, :H], p['nW1'][i, H:], p['nb1'][i][None],
                       p['nW2'][i], p['nb2'][i][None],
                       p['ln_g'][None], p['ln_b'][None])
        x = x + xo[:N, :3]
    return h, x


# speedup vs baseline: 5.0688x; 1.2143x over previous
import dataclasses
import functools
import math

import jax
import jax.numpy as jnp
from jax import lax
from jax.experimental import pallas as pl
from jax.experimental.pallas import tpu as pltpu
from jax.experimental.pallas import tpu_sc as plsc

N = 10000
E = 160000
H = 128

_NC = 2
_NS = 16
_NW = _NC * _NS
_C = 128
_EROWS = 1280
_EPAD = _EROWS * _C
_RPW = _EROWS // _NW
_NP = 10240

_BE = 1024
_BN = 1000

_sc_mesh = plsc.VectorSubcoreMesh(core_axis_name="c", subcore_axis_name="s")

_sc_cp = pltpu.CompilerParams()
if "needs_layout_passes" in pltpu.CompilerParams.__dataclass_fields__:
    _sc_cp = dataclasses.replace(_sc_cp, needs_layout_passes=False)


def _sc_gather(h2b, x4, rowg, colg):
    @functools.partial(
        pl.kernel, mesh=_sc_mesh,
        out_type=[jax.ShapeDtypeStruct((_EPAD, H), jnp.float32),
                  jax.ShapeDtypeStruct((_EPAD, H), jnp.float32),
                  jax.ShapeDtypeStruct((8, _EPAD), jnp.float32)],
        scratch_types=[pltpu.VMEM((_RPW, _C), jnp.int32),
                       pltpu.VMEM((_RPW, _C), jnp.int32),
                       pltpu.VMEM((_C, H), jnp.float32),
                       pltpu.VMEM((_C, H), jnp.float32),
                       pltpu.VMEM((4 * N,), jnp.float32),
                       pltpu.VMEM((8, _C), jnp.float32),
                       pltpu.SemaphoreType.DMA],
        compiler_params=_sc_cp,
    )
    def k(h_hbm, x_hbm, ri_hbm, ci_hbm, hr_hbm, hc_hbm, rel_hbm,
          ribuf, cibuf, hbuf, cbuf, x4v, relbuf, sem):
        wid = lax.axis_index("s") * _NC + lax.axis_index("c")
        base = wid * _RPW
        pltpu.sync_copy(ri_hbm.at[pl.ds(base, _RPW)], ribuf)
        pltpu.sync_copy(ci_hbm.at[pl.ds(base, _RPW)], cibuf)
        pltpu.sync_copy(x_hbm, x4v)
        zero16 = jnp.zeros((16,), jnp.float32)
        for r in range(4, 8):
            for kk in range(8):
                relbuf[r, pl.ds(kk * 16, 16)] = zero16

        @pl.loop(0, _RPW)
        def _(j):
            dh = pltpu.async_copy(h_hbm.at[ribuf.at[j]], hbuf, sem)
            dc = pltpu.async_copy(h_hbm.at[cibuf.at[j]], cbuf, sem)
            for kk in range(8):
                sl = pl.ds(kk * 16, 16)
                ir = ribuf[j, sl] * 4
                ic = cibuf[j, sl] * 4
                d2 = zero16
                for comp in range(3):
                    cidx = jnp.full((16,), comp, jnp.int32)
                    vr = plsc.load_gather(x4v, [ir + cidx])
                    vc = plsc.load_gather(x4v, [ic + cidx])
                    rr = vr - vc
                    relbuf[comp, sl] = rr
                    d2 = d2 + rr * rr
                relbuf[3, sl] = d2
            e0 = (base + j) * _C
            dh.wait()
            dc.wait()
            pltpu.sync_copy(hbuf, hr_hbm.at[pl.ds(e0, _C)])
            pltpu.sync_copy(cbuf, hc_hbm.at[pl.ds(e0, _C)])
            pltpu.sync_copy(relbuf, rel_hbm.at[:, pl.ds(e0, _C)])

    return k(h2b, x4, rowg, colg)


def _sc_scatter(msg, xmsg, rows, zm, zx):
    rps = _NP // _NS
    rpc = _EROWS // _NC
    rps_e = rpc // _NS

    nx = _NP * 4
    nxs = nx // _NS

    @functools.partial(
        pl.kernel, mesh=_sc_mesh,
        out_type=[jax.ShapeDtypeStruct((_NC, _NP, H), jnp.float32),
                  jax.ShapeDtypeStruct((_NC, nx), jnp.float32)],
        scratch_types=[pltpu.VMEM((rps_e, _C), jnp.int32),
                       pltpu.VMEM((_C, H), jnp.float32),
                       pltpu.VMEM((_C, H), jnp.float32),
                       pltpu.VMEM((4, _C), jnp.float32),
                       pltpu.VMEM((4, _C), jnp.float32),
                       pltpu.VMEM((4, _C), jnp.int32),
                       pltpu.VMEM_SHARED((_NP, H), jnp.float32),
                       pltpu.VMEM_SHARED((nx,), jnp.float32),
                       pltpu.SemaphoreType.DMA],
        compiler_params=_sc_cp,
    )
    def k(m_hbm, xm_hbm, ri_hbm, zm_hbm, zx_hbm, mo_hbm, xo_hbm,
          ibuf, mbuf0, mbuf1, xbuf0, xbuf1, ixbuf, macc, xacc, sem):
        c = lax.axis_index("c")
        s = lax.axis_index("s")
        pltpu.sync_copy(zm_hbm, macc.at[pl.ds(s * rps, rps)])
        pltpu.sync_copy(zx_hbm, xacc.at[pl.ds(s * nxs, nxs)])
        plsc.subcore_barrier()
        base = c * rpc + s * rps_e
        pltpu.sync_copy(ri_hbm.at[pl.ds(base, rps_e)], ibuf)

        def xscat(j, xbuf):
            for r in range(3):
                for g in range(8):
                    sl = pl.ds(g * 16, 16)
                    ixbuf[r, sl] = ibuf[j, sl] * 4 + r
            for r in range(3):
                pltpu.sync_copy(xbuf.at[r], xacc.at[ixbuf.at[r]],
                                add=True)

        @pl.loop(0, rps_e // 2)
        def _(jp):
            j0 = jp * 2
            e0 = (base + j0) * _C
            d0 = pltpu.async_copy(m_hbm.at[pl.ds(e0, _C)], mbuf0, sem)
            dx0 = pltpu.async_copy(xm_hbm.at[:, pl.ds(e0, _C)], xbuf0, sem)
            d1 = pltpu.async_copy(m_hbm.at[pl.ds(e0 + _C, _C)], mbuf1, sem)
            dx1 = pltpu.async_copy(xm_hbm.at[:, pl.ds(e0 + _C, _C)], xbuf1,
                                   sem)
            d0.wait()
            dx0.wait()
            pltpu.sync_copy(mbuf0, macc.at[ibuf.at[j0]], add=True)
            xscat(j0, xbuf0)
            d1.wait()
            dx1.wait()
            pltpu.sync_copy(mbuf1, macc.at[ibuf.at[j0 + 1]], add=True)
            xscat(j0 + 1, xbuf1)

        plsc.subcore_barrier()
        pltpu.sync_copy(macc.at[pl.ds(s * rps, rps)],
                        mo_hbm.at[c, pl.ds(s * rps, rps)])
        pltpu.sync_copy(xacc.at[pl.ds(s * nxs, nxs)],
                        xo_hbm.at[c, pl.ds(s * nxs, nxs)])

    return k(msg, xmsg, rows, zm, zx)


def _silu(v):
    return v * jax.nn.sigmoid(v)


def _edge_body(hr, hc, rel8, eye8, msk8, w1h1, w1h2, w1d, b1, w2, b2,
               aw, ab, cw1, cb1, cw2, msg, xout):
    bf = jnp.bfloat16
    f32 = jnp.float32
    r8 = lax.dot_general(rel8[...], eye8[...], (((0,), (0,)), ((), ())),
                         preferred_element_type=f32)
    dist = jnp.sqrt(r8[:, 3:4])
    t1 = (jnp.dot(hr[...].astype(bf), w1h1[...],
                  preferred_element_type=f32)
          + jnp.dot(hc[...].astype(bf), w1h2[...],
                    preferred_element_type=f32)
          + dist * w1d[...] + b1[...]).astype(bf)
    t1 = _silu(t1)
    m = (jnp.dot(t1, w2[...], preferred_element_type=f32)
         + b2[...]).astype(bf)
    m = _silu(m)
    att = jax.nn.sigmoid(jnp.dot(m, aw[...], preferred_element_type=f32)
                         + ab[...])
    msg[...] = att * m.astype(f32)
    c1 = _silu((jnp.dot(m, cw1[...], preferred_element_type=f32)
                + cb1[...]).astype(bf))
    cwT = lax.dot_general(cw2[...], c1, (((1,), (1,)), ((), ())),
                          preferred_element_type=f32)
    distT = jnp.sqrt(rel8[3:4, :])
    xout[...] = (cwT * rel8[0:4, :] / (distT + 1e-8)) * msk8[...]


def _edge_block_call(hr, hc, rel8, eye8, msk8, wts):
    grid = _EPAD // _BE
    full = lambda s: pl.BlockSpec(s, lambda i: (0,) * len(s))
    eb = lambda d: pl.BlockSpec((_BE, d), lambda i: (i, 0))
    return pl.pallas_call(
        _edge_body,
        grid=(grid,),
        in_specs=[eb(H), eb(H), pl.BlockSpec((8, _BE), lambda i: (0, i)),
                  full((8, 8)), full((4, 1)),
                  full((H, H)), full((H, H)), full((1, H)), full((1, H)),
                  full((H, H)), full((1, H)), full((H, 1)), full((1, 1)),
                  full((H, H)), full((1, H)), full((1, H))],
        out_specs=[eb(H), pl.BlockSpec((4, _BE), lambda i: (0, i))],
        out_shape=[jax.ShapeDtypeStruct((_EPAD, H), jnp.float32),
                   jax.ShapeDtypeStruct((4, _EPAD), jnp.float32)],
        compiler_params=pltpu.CompilerParams(
            dimension_semantics=("parallel",)),
    )(hr, hc, rel8, eye8, msk8, *wts)


def _node_body(ln, h, m0, m1, w1a, w1b, b1, w2, b2, g, bv, out):
    mi = m0[0] + m1[0]
    t = (jnp.dot(h[...], w1a[...], preferred_element_type=jnp.float32)
         + jnp.dot(mi, w1b[...], preferred_element_type=jnp.float32)
         + b1[...])
    t = _silu(t)
    hn = jnp.dot(t, w2[...], preferred_element_type=jnp.float32) + b2[...]
    hnew = h[...] + hn
    if ln:
        mu = jnp.mean(hnew, axis=-1, keepdims=True)
        va = jnp.mean((hnew - mu) ** 2, axis=-1, keepdims=True)
        hnew = (hnew - mu) / jnp.sqrt(va + 1e-5) * g[...] + bv[...]
    out[...] = hnew


def _node_call(ln, h, mo, w1a, w1b, b1, w2, b2, g, bv):
    grid = N // _BN
    full = lambda s: pl.BlockSpec(s, lambda i: (0,) * len(s))
    nb = pl.BlockSpec((_BN, H), lambda i: (i, 0))
    m0 = pl.BlockSpec((1, _BN, H), lambda i: (0, i, 0))
    m1 = pl.BlockSpec((1, _BN, H), lambda i: (1, i, 0))
    return pl.pallas_call(
        functools.partial(_node_body, ln),
        grid=(grid,),
        in_specs=[nb, m0, m1, full((H, H)), full((H, H)), full((1, H)),
                  full((H, H)), full((1, H)), full((1, H)), full((1, H))],
        out_specs=nb,
        out_shape=jax.ShapeDtypeStruct((N, H), jnp.float32),
        compiler_params=pltpu.CompilerParams(
            dimension_semantics=("parallel",)),
    )(h, mo, mo, w1a, w1b, b1, w2, b2, g, bv)


def _precompute(t, af, dc, cc, p):
    silu = jax.nn.silu
    half = H // 2
    freqs = jnp.exp(jnp.arange(half, dtype=jnp.float32)
                    * (-(math.log(10000.0) / (half - 1))))
    te = t.astype(jnp.float32)[:, None] * freqs[None, :]
    temb = jnp.concatenate([jnp.sin(te), jnp.cos(te)], axis=-1)
    a = silu(af @ p['ce_aW1'] + p['ce_ab1']) @ p['ce_aW2'] + p['ce_ab2']
    a_emb = a.mean(axis=0, keepdims=True)
    d = silu(dc @ p['ce_dW1'] + p['ce_db1']) @ p['ce_dW2'] + p['ce_db2']
    d_emb = d.mean(axis=0, keepdims=True)
    c = silu(cc @ p['ce_cW1'] + p['ce_cb1']) @ p['ce_cW2'] + p['ce_cb2']
    c_emb = c.mean(axis=0, keepdims=True)
    comb = jnp.concatenate([a_emb, d_emb, c_emb], axis=-1)
    z = comb @ p['ce_fW1'] + p['ce_fb1']
    mu = z.mean(axis=-1, keepdims=True)
    va = ((z - mu) ** 2).mean(axis=-1, keepdims=True)
    z = (z - mu) / jnp.sqrt(va + 1e-5) * p['ce_fg'] + p['ce_fbe']
    cond = silu(z) @ p['ce_fW2'] + p['ce_fb2']
    tproj = temb @ p['tpW'] + p['tpb']
    cproj = cond @ p['cpW'] + p['cpb']
    return tproj + cproj


def kernel(h, x, edge_index, t, anchor_features, distance_constraints,
           coordination_constraints, params):
    p = params
    npad = _EPAD - E
    gpad = (jnp.arange(npad, dtype=jnp.int32) * 37) % N
    rowg = jnp.concatenate([edge_index[0], gpad]).reshape(_EROWS, _C)
    colg = jnp.concatenate([edge_index[1], gpad]).reshape(_EROWS, _C)
    spad = N + (jnp.arange(npad, dtype=jnp.int32) % (_NP - N))
    rows = jnp.concatenate([edge_index[0], spad]).reshape(_EROWS, _C)
    zm = jnp.zeros((_NP // _NS, H), jnp.float32)
    zx = jnp.zeros((_NP * 4 // _NS,), jnp.float32)
    eye8 = jnp.eye(8, dtype=jnp.float32)
    msk8 = jnp.array([[1.], [1.], [1.], [0.]], jnp.float32)

    h = h + _precompute(t, anchor_features, distance_constraints,
                        coordination_constraints, p)
    for i in range(2):
        x4 = jnp.pad(x, ((0, 0), (0, 1))).reshape(-1)
        hr, hc, rel8 = _sc_gather(h, x4, rowg, colg)
        bf = jnp.bfloat16
        wts = (p['eW1'][i, :H].astype(bf), p['eW1'][i, H:2 * H].astype(bf),
               p['eW1'][i, 2 * H:], p['eb1'][i][None],
               p['eW2'][i].astype(bf), p['eb2'][i][None],
               p['aW'][i].astype(bf), p['ab'][i][None],
               p['cW1'][i].astype(bf), p['cb1'][i][None],
               p['cW2'][i].T.astype(bf))
        msg, xmsg = _edge_block_call(hr, hc, rel8, eye8, msk8, wts)
        mo, xo = _sc_scatter(msg, xmsg, rows, zm, zx)
        h = _node_call(i == 1, h, mo,
                       p['nW1'][i, :H], p['nW1'][i, H:], p['nb1'][i][None],
                       p['nW2'][i], p['nb2'][i][None],
                       p['ln_g'][None], p['ln_b'][None])
        xr = (xo[0] + xo[1]).reshape(_NP, 4)
        x = x + xr[:N, :3]
    return h, x


# speedup vs baseline: 5.3349x; 1.0525x over previous
import dataclasses
import functools
import math

import jax
import jax.numpy as jnp
from jax import lax
from jax.experimental import pallas as pl
from jax.experimental.pallas import tpu as pltpu
from jax.experimental.pallas import tpu_sc as plsc

N = 10000
E = 160000
H = 128

_NC = 2
_NS = 16
_NW = _NC * _NS
_C = 128
_EROWS = 1280
_EPAD = _EROWS * _C
_RPW = _EROWS // _NW
_NP = 10240

_BE = 1024
_BN = 1000

_sc_mesh = plsc.VectorSubcoreMesh(core_axis_name="c", subcore_axis_name="s")

_sc_cp = pltpu.CompilerParams()
if "needs_layout_passes" in pltpu.CompilerParams.__dataclass_fields__:
    _sc_cp = dataclasses.replace(_sc_cp, needs_layout_passes=False)


def _sc_gather(h2b, x4, rowg, colg):
    @functools.partial(
        pl.kernel, mesh=_sc_mesh,
        out_type=[jax.ShapeDtypeStruct((_EPAD, H), jnp.float32),
                  jax.ShapeDtypeStruct((_EPAD, H), jnp.float32),
                  jax.ShapeDtypeStruct((8, _EPAD), jnp.float32)],
        scratch_types=[pltpu.VMEM((_RPW, _C), jnp.int32),
                       pltpu.VMEM((_RPW, _C), jnp.int32),
                       pltpu.VMEM((_C, H), jnp.float32),
                       pltpu.VMEM((_C, H), jnp.float32),
                       pltpu.VMEM((_C, H), jnp.float32),
                       pltpu.VMEM((_C, H), jnp.float32),
                       pltpu.VMEM((4 * N,), jnp.float32),
                       pltpu.VMEM((8, _C), jnp.float32),
                       pltpu.VMEM((8, _C), jnp.float32),
                       pltpu.SemaphoreType.DMA,
                       pltpu.SemaphoreType.DMA],
        compiler_params=_sc_cp,
    )
    def k(h_hbm, x_hbm, ri_hbm, ci_hbm, hr_hbm, hc_hbm, rel_hbm,
          ribuf, cibuf, hbuf0, cbuf0, hbuf1, cbuf1, x4v, relbuf0, relbuf1,
          semg, semw):
        wid = lax.axis_index("s") * _NC + lax.axis_index("c")
        base = wid * _RPW
        pltpu.sync_copy(ri_hbm.at[pl.ds(base, _RPW)], ribuf)
        pltpu.sync_copy(ci_hbm.at[pl.ds(base, _RPW)], cibuf)
        pltpu.sync_copy(x_hbm, x4v)
        zero16 = jnp.zeros((16,), jnp.float32)
        for rb in (relbuf0, relbuf1):
            for r in range(4, 8):
                for kk in range(8):
                    rb[r, pl.ds(kk * 16, 16)] = zero16

        def relcompute(j, rb):
            for kk in range(8):
                sl = pl.ds(kk * 16, 16)
                ir = ribuf[j, sl] * 4
                ic = cibuf[j, sl] * 4
                d2 = zero16
                for comp in range(3):
                    cidx = jnp.full((16,), comp, jnp.int32)
                    vr = plsc.load_gather(x4v, [ir + cidx])
                    vc = plsc.load_gather(x4v, [ic + cidx])
                    rr = vr - vc
                    rb[comp, sl] = rr
                    d2 = d2 + rr * rr
                rb[3, sl] = d2

        def drain_writes():
            pltpu.make_async_copy(hr_hbm.at[pl.ds(0, _C)], hbuf0,
                                  semw).wait()
            pltpu.make_async_copy(hc_hbm.at[pl.ds(0, _C)], cbuf0,
                                  semw).wait()
            pltpu.make_async_copy(rel_hbm.at[:, pl.ds(0, _C)], relbuf0,
                                  semw).wait()

        @pl.loop(0, _RPW // 2)
        def _(jp):
            j0 = jp * 2

            @pl.when(jp > 0)
            def _():
                drain_writes()
                drain_writes()

            g0a = pltpu.async_copy(h_hbm.at[ribuf.at[j0]], hbuf0, semg)
            g0b = pltpu.async_copy(h_hbm.at[cibuf.at[j0]], cbuf0, semg)
            g1a = pltpu.async_copy(h_hbm.at[ribuf.at[j0 + 1]], hbuf1, semg)
            g1b = pltpu.async_copy(h_hbm.at[cibuf.at[j0 + 1]], cbuf1, semg)
            relcompute(j0, relbuf0)
            e0 = (base + j0) * _C
            g0a.wait()
            g0b.wait()
            pltpu.async_copy(hbuf0, hr_hbm.at[pl.ds(e0, _C)], semw)
            pltpu.async_copy(cbuf0, hc_hbm.at[pl.ds(e0, _C)], semw)
            pltpu.async_copy(relbuf0, rel_hbm.at[:, pl.ds(e0, _C)], semw)
            relcompute(j0 + 1, relbuf1)
            g1a.wait()
            g1b.wait()
            pltpu.async_copy(hbuf1, hr_hbm.at[pl.ds(e0 + _C, _C)], semw)
            pltpu.async_copy(cbuf1, hc_hbm.at[pl.ds(e0 + _C, _C)], semw)
            pltpu.async_copy(relbuf1, rel_hbm.at[:, pl.ds(e0 + _C, _C)],
                             semw)

        drain_writes()
        drain_writes()

    return k(h2b, x4, rowg, colg)


def _sc_scatter(msg, xmsg, rows, zm, zx):
    rps = _NP // _NS
    rpc = _EROWS // _NC
    rps_e = rpc // _NS

    nx = _NP * 4
    nxs = nx // _NS

    @functools.partial(
        pl.kernel, mesh=_sc_mesh,
        out_type=[jax.ShapeDtypeStruct((_NC, _NP, H), jnp.float32),
                  jax.ShapeDtypeStruct((_NC, nx), jnp.float32)],
        scratch_types=[pltpu.VMEM((rps_e, _C), jnp.int32),
                       pltpu.VMEM((_C, H), jnp.float32),
                       pltpu.VMEM((_C, H), jnp.float32),
                       pltpu.VMEM((4, _C), jnp.float32),
                       pltpu.VMEM((4, _C), jnp.float32),
                       pltpu.VMEM((4, _C), jnp.int32),
                       pltpu.VMEM_SHARED((_NP, H), jnp.float32),
                       pltpu.VMEM_SHARED((nx,), jnp.float32),
                       pltpu.SemaphoreType.DMA],
        compiler_params=_sc_cp,
    )
    def k(m_hbm, xm_hbm, ri_hbm, zm_hbm, zx_hbm, mo_hbm, xo_hbm,
          ibuf, mbuf0, mbuf1, xbuf0, xbuf1, ixbuf, macc, xacc, sem):
        c = lax.axis_index("c")
        s = lax.axis_index("s")
        pltpu.sync_copy(zm_hbm, macc.at[pl.ds(s * rps, rps)])
        pltpu.sync_copy(zx_hbm, xacc.at[pl.ds(s * nxs, nxs)])
        plsc.subcore_barrier()
        base = c * rpc + s * rps_e
        pltpu.sync_copy(ri_hbm.at[pl.ds(base, rps_e)], ibuf)

        def xscat(j, xbuf):
            for r in range(3):
                for g in range(8):
                    sl = pl.ds(g * 16, 16)
                    ixbuf[r, sl] = ibuf[j, sl] * 4 + r
            for r in range(3):
                pltpu.sync_copy(xbuf.at[r], xacc.at[ixbuf.at[r]],
                                add=True)

        @pl.loop(0, rps_e // 2)
        def _(jp):
            j0 = jp * 2
            e0 = (base + j0) * _C
            d0 = pltpu.async_copy(m_hbm.at[pl.ds(e0, _C)], mbuf0, sem)
            dx0 = pltpu.async_copy(xm_hbm.at[:, pl.ds(e0, _C)], xbuf0, sem)
            d1 = pltpu.async_copy(m_hbm.at[pl.ds(e0 + _C, _C)], mbuf1, sem)
            dx1 = pltpu.async_copy(xm_hbm.at[:, pl.ds(e0 + _C, _C)], xbuf1,
                                   sem)
            d0.wait()
            dx0.wait()
            pltpu.sync_copy(mbuf0, macc.at[ibuf.at[j0]], add=True)
            xscat(j0, xbuf0)
            d1.wait()
            dx1.wait()
            pltpu.sync_copy(mbuf1, macc.at[ibuf.at[j0 + 1]], add=True)
            xscat(j0 + 1, xbuf1)

        plsc.subcore_barrier()
        pltpu.sync_copy(macc.at[pl.ds(s * rps, rps)],
                        mo_hbm.at[c, pl.ds(s * rps, rps)])
        pltpu.sync_copy(xacc.at[pl.ds(s * nxs, nxs)],
                        xo_hbm.at[c, pl.ds(s * nxs, nxs)])

    return k(msg, xmsg, rows, zm, zx)


def _silu(v):
    return v * jax.nn.sigmoid(v)


def _edge_body(hr, hc, rel8, eye8, msk8, w1cat, w1d, b1, w2, b2,
               aw, ab, cw1, cb1, cw2, msg, xout):
    bf = jnp.bfloat16
    f32 = jnp.float32
    r8 = lax.dot_general(rel8[...], eye8[...], (((0,), (0,)), ((), ())),
                         preferred_element_type=f32)
    distb = jnp.sqrt(r8[:, 3:4]).astype(bf)
    hh = jnp.concatenate([hr[...].astype(bf), hc[...].astype(bf)], axis=1)
    t1 = (jnp.dot(hh, w1cat[...], preferred_element_type=f32).astype(bf)
          + distb * w1d[...] + b1[...])
    t1 = _silu(t1)
    m = jnp.dot(t1, w2[...], preferred_element_type=f32).astype(bf) + b2[...]
    m = _silu(m)
    att = jax.nn.sigmoid(jnp.dot(m, aw[...], preferred_element_type=f32)
                         + ab[...])
    msg[...] = att * m.astype(f32)
    c1 = _silu(jnp.dot(m, cw1[...], preferred_element_type=f32).astype(bf)
               + cb1[...])
    cwT = lax.dot_general(cw2[...], c1, (((1,), (1,)), ((), ())),
                          preferred_element_type=f32)
    distT = jnp.sqrt(rel8[3:4, :])
    xout[...] = (cwT * rel8[0:4, :] / (distT + 1e-8)) * msk8[...]


def _edge_block_call(hr, hc, rel8, eye8, msk8, wts):
    grid = _EPAD // _BE
    full = lambda s: pl.BlockSpec(s, lambda i: (0,) * len(s))
    eb = lambda d: pl.BlockSpec((_BE, d), lambda i: (i, 0))
    return pl.pallas_call(
        _edge_body,
        grid=(grid,),
        in_specs=[eb(H), eb(H), pl.BlockSpec((8, _BE), lambda i: (0, i)),
                  full((8, 8)), full((4, 1)),
                  full((2 * H, H)), full((1, H)), full((1, H)),
                  full((H, H)), full((1, H)), full((H, 1)), full((1, 1)),
                  full((H, H)), full((1, H)), full((1, H))],
        out_specs=[eb(H), pl.BlockSpec((4, _BE), lambda i: (0, i))],
        out_shape=[jax.ShapeDtypeStruct((_EPAD, H), jnp.float32),
                   jax.ShapeDtypeStruct((4, _EPAD), jnp.float32)],
        compiler_params=pltpu.CompilerParams(
            dimension_semantics=("parallel",)),
    )(hr, hc, rel8, eye8, msk8, *wts)


def _node_body(ln, h, m0, m1, w1a, w1b, b1, w2, b2, g, bv, out):
    mi = m0[0] + m1[0]
    t = (jnp.dot(h[...], w1a[...], preferred_element_type=jnp.float32)
         + jnp.dot(mi, w1b[...], preferred_element_type=jnp.float32)
         + b1[...])
    t = _silu(t)
    hn = jnp.dot(t, w2[...], preferred_element_type=jnp.float32) + b2[...]
    hnew = h[...] + hn
    if ln:
        mu = jnp.mean(hnew, axis=-1, keepdims=True)
        va = jnp.mean((hnew - mu) ** 2, axis=-1, keepdims=True)
        hnew = (hnew - mu) / jnp.sqrt(va + 1e-5) * g[...] + bv[...]
    out[...] = hnew


def _node_call(ln, h, mo, w1a, w1b, b1, w2, b2, g, bv):
    grid = N // _BN
    full = lambda s: pl.BlockSpec(s, lambda i: (0,) * len(s))
    nb = pl.BlockSpec((_BN, H), lambda i: (i, 0))
    m0 = pl.BlockSpec((1, _BN, H), lambda i: (0, i, 0))
    m1 = pl.BlockSpec((1, _BN, H), lambda i: (1, i, 0))
    return pl.pallas_call(
        functools.partial(_node_body, ln),
        grid=(grid,),
        in_specs=[nb, m0, m1, full((H, H)), full((H, H)), full((1, H)),
                  full((H, H)), full((1, H)), full((1, H)), full((1, H))],
        out_specs=nb,
        out_shape=jax.ShapeDtypeStruct((N, H), jnp.float32),
        compiler_params=pltpu.CompilerParams(
            dimension_semantics=("parallel",)),
    )(h, mo, mo, w1a, w1b, b1, w2, b2, g, bv)


def _precompute(t, af, dc, cc, p):
    silu = jax.nn.silu
    half = H // 2
    freqs = jnp.exp(jnp.arange(half, dtype=jnp.float32)
                    * (-(math.log(10000.0) / (half - 1))))
    te = t.astype(jnp.float32)[:, None] * freqs[None, :]
    temb = jnp.concatenate([jnp.sin(te), jnp.cos(te)], axis=-1)
    a = silu(af @ p['ce_aW1'] + p['ce_ab1']) @ p['ce_aW2'] + p['ce_ab2']
    a_emb = a.mean(axis=0, keepdims=True)
    d = silu(dc @ p['ce_dW1'] + p['ce_db1']) @ p['ce_dW2'] + p['ce_db2']
    d_emb = d.mean(axis=0, keepdims=True)
    c = silu(cc @ p['ce_cW1'] + p['ce_cb1']) @ p['ce_cW2'] + p['ce_cb2']
    c_emb = c.mean(axis=0, keepdims=True)
    comb = jnp.concatenate([a_emb, d_emb, c_emb], axis=-1)
    z = comb @ p['ce_fW1'] + p['ce_fb1']
    mu = z.mean(axis=-1, keepdims=True)
    va = ((z - mu) ** 2).mean(axis=-1, keepdims=True)
    z = (z - mu) / jnp.sqrt(va + 1e-5) * p['ce_fg'] + p['ce_fbe']
    cond = silu(z) @ p['ce_fW2'] + p['ce_fb2']
    tproj = temb @ p['tpW'] + p['tpb']
    cproj = cond @ p['cpW'] + p['cpb']
    return tproj + cproj


def kernel(h, x, edge_index, t, anchor_features, distance_constraints,
           coordination_constraints, params):
    p = params
    npad = _EPAD - E
    gpad = (jnp.arange(npad, dtype=jnp.int32) * 37) % N
    rowg = jnp.concatenate([edge_index[0], gpad]).reshape(_EROWS, _C)
    colg = jnp.concatenate([edge_index[1], gpad]).reshape(_EROWS, _C)
    spad = N + (jnp.arange(npad, dtype=jnp.int32) % (_NP - N))
    rows = jnp.concatenate([edge_index[0], spad]).reshape(_EROWS, _C)
    zm = jnp.zeros((_NP // _NS, H), jnp.float32)
    zx = jnp.zeros((_NP * 4 // _NS,), jnp.float32)
    eye8 = jnp.eye(8, dtype=jnp.float32)
    msk8 = jnp.array([[1.], [1.], [1.], [0.]], jnp.float32)

    h = h + _precompute(t, anchor_features, distance_constraints,
                        coordination_constraints, p)
    for i in range(2):
        x4 = jnp.pad(x, ((0, 0), (0, 1))).reshape(-1)
        hr, hc, rel8 = _sc_gather(h, x4, rowg, colg)
        bf = jnp.bfloat16
        wts = (p['eW1'][i, :2 * H].astype(bf),
               p['eW1'][i, 2 * H:].astype(bf), p['eb1'][i][None].astype(bf),
               p['eW2'][i].astype(bf), p['eb2'][i][None].astype(bf),
               p['aW'][i].astype(bf), p['ab'][i][None],
               p['cW1'][i].astype(bf), p['cb1'][i][None].astype(bf),
               p['cW2'][i].T.astype(bf))
        msg, xmsg = _edge_block_call(hr, hc, rel8, eye8, msk8, wts)
        mo, xo = _sc_scatter(msg, xmsg, rows, zm, zx)
        h = _node_call(i == 1, h, mo,
                       p['nW1'][i, :H], p['nW1'][i, H:], p['nb1'][i][None],
                       p['nW2'][i], p['nb2'][i][None],
                       p['ln_g'][None], p['ln_b'][None])
        xr = (xo[0] + xo[1]).reshape(_NP, 4)
        x = x + xr[:N, :3]
    return h, x


# speedup vs baseline: 5.9058x; 1.1070x over previous
import dataclasses
import functools
import math

import jax
import jax.numpy as jnp
from jax import lax
from jax.experimental import pallas as pl
from jax.experimental.pallas import tpu as pltpu
from jax.experimental.pallas import tpu_sc as plsc

N = 10000
E = 160000
H = 128

_NC = 2
_NS = 16
_NW = _NC * _NS
_C = 128
_EROWS = 1280
_EPAD = _EROWS * _C
_RPW = _EROWS // _NW
_NP = 10240

_BE = 1024
_BN = 1000

_sc_mesh = plsc.VectorSubcoreMesh(core_axis_name="c", subcore_axis_name="s")

_sc_cp = pltpu.CompilerParams()
if "needs_layout_passes" in pltpu.CompilerParams.__dataclass_fields__:
    _sc_cp = dataclasses.replace(_sc_cp, needs_layout_passes=False)


def _sc_gather(h2b, x4, rowg, colg):
    nrows = rowg.shape[0] // _C
    rpw = nrows // _NW
    npad = nrows * _C

    @functools.partial(
        pl.kernel, mesh=_sc_mesh,
        out_type=[jax.ShapeDtypeStruct((npad, H), jnp.float32),
                  jax.ShapeDtypeStruct((npad, H), jnp.float32),
                  jax.ShapeDtypeStruct((8, npad), jnp.float32)],
        scratch_types=[pltpu.VMEM((rpw, _C), jnp.int32),
                       pltpu.VMEM((rpw, _C), jnp.int32),
                       pltpu.VMEM((_C, H), jnp.float32),
                       pltpu.VMEM((_C, H), jnp.float32),
                       pltpu.VMEM((_C, H), jnp.float32),
                       pltpu.VMEM((_C, H), jnp.float32),
                       pltpu.VMEM((4 * N,), jnp.float32),
                       pltpu.VMEM((8, _C), jnp.float32),
                       pltpu.VMEM((8, _C), jnp.float32),
                       pltpu.SemaphoreType.DMA,
                       pltpu.SemaphoreType.DMA],
        compiler_params=_sc_cp,
    )
    def k(h_hbm, x_hbm, ri_hbm, ci_hbm, hr_hbm, hc_hbm, rel_hbm,
          ribuf, cibuf, hbuf0, cbuf0, hbuf1, cbuf1, x4v, relbuf0, relbuf1,
          semg, semw):
        wid = lax.axis_index("s") * _NC + lax.axis_index("c")
        base = wid * rpw
        idma = []
        for j in range(rpw):
            idma.append(pltpu.async_copy(
                ri_hbm.at[pl.ds((base + j) * _C, _C)], ribuf.at[j], semg))
            idma.append(pltpu.async_copy(
                ci_hbm.at[pl.ds((base + j) * _C, _C)], cibuf.at[j], semg))
        pltpu.sync_copy(x_hbm, x4v)
        for d in idma:
            d.wait()
        zero16 = jnp.zeros((16,), jnp.float32)
        for rb in (relbuf0, relbuf1):
            for r in range(4, 8):
                for kk in range(8):
                    rb[r, pl.ds(kk * 16, 16)] = zero16

        def relcompute(j, rb):
            for kk in range(8):
                sl = pl.ds(kk * 16, 16)
                ir = ribuf[j, sl] * 4
                ic = cibuf[j, sl] * 4
                d2 = zero16
                for comp in range(3):
                    cidx = jnp.full((16,), comp, jnp.int32)
                    vr = plsc.load_gather(x4v, [ir + cidx])
                    vc = plsc.load_gather(x4v, [ic + cidx])
                    rr = vr - vc
                    rb[comp, sl] = rr
                    d2 = d2 + rr * rr
                rb[3, sl] = d2

        def drain_writes():
            pltpu.make_async_copy(hr_hbm.at[pl.ds(0, _C)], hbuf0,
                                  semw).wait()
            pltpu.make_async_copy(hc_hbm.at[pl.ds(0, _C)], cbuf0,
                                  semw).wait()
            pltpu.make_async_copy(rel_hbm.at[:, pl.ds(0, _C)], relbuf0,
                                  semw).wait()

        @pl.loop(0, rpw // 2)
        def _(jp):
            j0 = jp * 2

            @pl.when(jp > 0)
            def _():
                drain_writes()
                drain_writes()

            g0a = pltpu.async_copy(h_hbm.at[ribuf.at[j0]], hbuf0, semg)
            g0b = pltpu.async_copy(h_hbm.at[cibuf.at[j0]], cbuf0, semg)
            g1a = pltpu.async_copy(h_hbm.at[ribuf.at[j0 + 1]], hbuf1, semg)
            g1b = pltpu.async_copy(h_hbm.at[cibuf.at[j0 + 1]], cbuf1, semg)
            relcompute(j0, relbuf0)
            e0 = (base + j0) * _C
            g0a.wait()
            g0b.wait()
            pltpu.async_copy(hbuf0, hr_hbm.at[pl.ds(e0, _C)], semw)
            pltpu.async_copy(cbuf0, hc_hbm.at[pl.ds(e0, _C)], semw)
            pltpu.async_copy(relbuf0, rel_hbm.at[:, pl.ds(e0, _C)], semw)
            relcompute(j0 + 1, relbuf1)
            g1a.wait()
            g1b.wait()
            pltpu.async_copy(hbuf1, hr_hbm.at[pl.ds(e0 + _C, _C)], semw)
            pltpu.async_copy(cbuf1, hc_hbm.at[pl.ds(e0 + _C, _C)], semw)
            pltpu.async_copy(relbuf1, rel_hbm.at[:, pl.ds(e0 + _C, _C)],
                             semw)

        drain_writes()
        drain_writes()

    return k(h2b, x4, rowg, colg)


def _sc_scatter(msg, xmsg, rows, zm, zx):
    rps = _NP // _NS
    rpc = rows.shape[0] // _C // _NC
    rps_e = rpc // _NS

    nx = _NP * 4
    nxs = nx // _NS

    @functools.partial(
        pl.kernel, mesh=_sc_mesh,
        out_type=[jax.ShapeDtypeStruct((_NC, _NP, H), jnp.float32),
                  jax.ShapeDtypeStruct((_NC, nx), jnp.float32)],
        scratch_types=[pltpu.VMEM((rps_e, _C), jnp.int32),
                       pltpu.VMEM((_C, H), jnp.float32),
                       pltpu.VMEM((_C, H), jnp.float32),
                       pltpu.VMEM((4, _C), jnp.float32),
                       pltpu.VMEM((4, _C), jnp.float32),
                       pltpu.VMEM((4, _C), jnp.int32),
                       pltpu.VMEM_SHARED((_NP, H), jnp.float32),
                       pltpu.VMEM_SHARED((nx,), jnp.float32),
                       pltpu.SemaphoreType.DMA],
        compiler_params=_sc_cp,
    )
    def k(m_hbm, xm_hbm, ri_hbm, zm_hbm, zx_hbm, mo_hbm, xo_hbm,
          ibuf, mbuf0, mbuf1, xbuf0, xbuf1, ixbuf, macc, xacc, sem):
        c = lax.axis_index("c")
        s = lax.axis_index("s")
        pltpu.sync_copy(zm_hbm, macc.at[pl.ds(s * rps, rps)])
        pltpu.sync_copy(zx_hbm, xacc.at[pl.ds(s * nxs, nxs)])
        plsc.subcore_barrier()
        base = c * rpc + s * rps_e
        idma = []
        for j in range(rps_e):
            idma.append(pltpu.async_copy(
                ri_hbm.at[pl.ds((base + j) * _C, _C)], ibuf.at[j], sem))
        for d in idma:
            d.wait()

        def xscat(j, xbuf):
            for r in range(3):
                for g in range(8):
                    sl = pl.ds(g * 16, 16)
                    ixbuf[r, sl] = ibuf[j, sl] * 4 + r
            for r in range(3):
                pltpu.sync_copy(xbuf.at[r], xacc.at[ixbuf.at[r]],
                                add=True)

        @pl.loop(0, rps_e // 2)
        def _(jp):
            j0 = jp * 2
            e0 = (base + j0) * _C
            d0 = pltpu.async_copy(m_hbm.at[pl.ds(e0, _C)], mbuf0, sem)
            dx0 = pltpu.async_copy(xm_hbm.at[:, pl.ds(e0, _C)], xbuf0, sem)
            d1 = pltpu.async_copy(m_hbm.at[pl.ds(e0 + _C, _C)], mbuf1, sem)
            dx1 = pltpu.async_copy(xm_hbm.at[:, pl.ds(e0 + _C, _C)], xbuf1,
                                   sem)
            d0.wait()
            dx0.wait()
            pltpu.sync_copy(mbuf0, macc.at[ibuf.at[j0]], add=True)
            xscat(j0, xbuf0)
            d1.wait()
            dx1.wait()
            pltpu.sync_copy(mbuf1, macc.at[ibuf.at[j0 + 1]], add=True)
            xscat(j0 + 1, xbuf1)

        plsc.subcore_barrier()
        pltpu.sync_copy(macc.at[pl.ds(s * rps, rps)],
                        mo_hbm.at[c, pl.ds(s * rps, rps)])
        pltpu.sync_copy(xacc.at[pl.ds(s * nxs, nxs)],
                        xo_hbm.at[c, pl.ds(s * nxs, nxs)])

    return k(msg, xmsg, rows, zm, zx)


def _silu(v):
    return v * jax.nn.sigmoid(v)


def _edge_body(hr, hc, rel8, eye8, msk8, w1cat, w1d, b1, w2, b2,
               aw, ab, cw1, cb1, cw2, msg, xout):
    bf = jnp.bfloat16
    f32 = jnp.float32
    r8 = lax.dot_general(rel8[...], eye8[...], (((0,), (0,)), ((), ())),
                         preferred_element_type=f32)
    distb = jnp.sqrt(r8[:, 3:4]).astype(bf)
    hh = jnp.concatenate([hr[...].astype(bf), hc[...].astype(bf)], axis=1)
    t1 = (jnp.dot(hh, w1cat[...], preferred_element_type=f32).astype(bf)
          + distb * w1d[...] + b1[...])
    t1 = _silu(t1)
    m = jnp.dot(t1, w2[...], preferred_element_type=f32).astype(bf) + b2[...]
    m = _silu(m)
    att = jax.nn.sigmoid(jnp.dot(m, aw[...], preferred_element_type=f32)
                         + ab[...])
    msg[...] = att * m.astype(f32)
    c1 = _silu(jnp.dot(m, cw1[...], preferred_element_type=f32).astype(bf)
               + cb1[...])
    cwT = lax.dot_general(cw2[...], c1, (((1,), (1,)), ((), ())),
                          preferred_element_type=f32)
    distT = jnp.sqrt(rel8[3:4, :])
    xout[...] = (cwT * rel8[0:4, :] / (distT + 1e-8)) * msk8[...]


def _edge_block_call(hr, hc, rel8, eye8, msk8, wts):
    n_e = hr.shape[0]
    grid = n_e // _BE
    full = lambda s: pl.BlockSpec(s, lambda i: (0,) * len(s))
    eb = lambda d: pl.BlockSpec((_BE, d), lambda i: (i, 0))
    return pl.pallas_call(
        _edge_body,
        grid=(grid,),
        in_specs=[eb(H), eb(H), pl.BlockSpec((8, _BE), lambda i: (0, i)),
                  full((8, 8)), full((4, 1)),
                  full((2 * H, H)), full((1, H)), full((1, H)),
                  full((H, H)), full((1, H)), full((H, 1)), full((1, 1)),
                  full((H, H)), full((1, H)), full((1, H))],
        out_specs=[eb(H), pl.BlockSpec((4, _BE), lambda i: (0, i))],
        out_shape=[jax.ShapeDtypeStruct((n_e, H), jnp.float32),
                   jax.ShapeDtypeStruct((4, n_e), jnp.float32)],
        compiler_params=pltpu.CompilerParams(
            dimension_semantics=("parallel",)),
    )(hr, hc, rel8, eye8, msk8, *wts)


def _node_body(ln, h, m0, m1, m2, m3, w1a, w1b, b1, w2, b2, g, bv, out):
    mi = (m0[0] + m1[0]) + (m2[0] + m3[0])
    t = (jnp.dot(h[...], w1a[...], preferred_element_type=jnp.float32)
         + jnp.dot(mi, w1b[...], preferred_element_type=jnp.float32)
         + b1[...])
    t = _silu(t)
    hn = jnp.dot(t, w2[...], preferred_element_type=jnp.float32) + b2[...]
    hnew = h[...] + hn
    if ln:
        mu = jnp.mean(hnew, axis=-1, keepdims=True)
        va = jnp.mean((hnew - mu) ** 2, axis=-1, keepdims=True)
        hnew = (hnew - mu) / jnp.sqrt(va + 1e-5) * g[...] + bv[...]
    out[...] = hnew


def _node_call(ln, h, moa, mob, w1a, w1b, b1, w2, b2, g, bv):
    grid = N // _BN
    full = lambda s: pl.BlockSpec(s, lambda i: (0,) * len(s))
    nb = pl.BlockSpec((_BN, H), lambda i: (i, 0))
    m0 = pl.BlockSpec((1, _BN, H), lambda i: (0, i, 0))
    m1 = pl.BlockSpec((1, _BN, H), lambda i: (1, i, 0))
    return pl.pallas_call(
        functools.partial(_node_body, ln),
        grid=(grid,),
        in_specs=[nb, m0, m1, m0, m1,
                  full((H, H)), full((H, H)), full((1, H)),
                  full((H, H)), full((1, H)), full((1, H)), full((1, H))],
        out_specs=nb,
        out_shape=jax.ShapeDtypeStruct((N, H), jnp.float32),
        compiler_params=pltpu.CompilerParams(
            dimension_semantics=("parallel",)),
    )(h, moa, moa, mob, mob, w1a, w1b, b1, w2, b2, g, bv)


def _precompute(t, af, dc, cc, p):
    silu = jax.nn.silu
    half = H // 2
    freqs = jnp.exp(jnp.arange(half, dtype=jnp.float32)
                    * (-(math.log(10000.0) / (half - 1))))
    te = t.astype(jnp.float32)[:, None] * freqs[None, :]
    temb = jnp.concatenate([jnp.sin(te), jnp.cos(te)], axis=-1)
    a = silu(af @ p['ce_aW1'] + p['ce_ab1']) @ p['ce_aW2'] + p['ce_ab2']
    a_emb = a.mean(axis=0, keepdims=True)
    d = silu(dc @ p['ce_dW1'] + p['ce_db1']) @ p['ce_dW2'] + p['ce_db2']
    d_emb = d.mean(axis=0, keepdims=True)
    c = silu(cc @ p['ce_cW1'] + p['ce_cb1']) @ p['ce_cW2'] + p['ce_cb2']
    c_emb = c.mean(axis=0, keepdims=True)
    comb = jnp.concatenate([a_emb, d_emb, c_emb], axis=-1)
    z = comb @ p['ce_fW1'] + p['ce_fb1']
    mu = z.mean(axis=-1, keepdims=True)
    va = ((z - mu) ** 2).mean(axis=-1, keepdims=True)
    z = (z - mu) / jnp.sqrt(va + 1e-5) * p['ce_fg'] + p['ce_fbe']
    cond = silu(z) @ p['ce_fW2'] + p['ce_fb2']
    tproj = temb @ p['tpW'] + p['tpb']
    cproj = cond @ p['cpW'] + p['cpb']
    return tproj + cproj


def kernel(h, x, edge_index, t, anchor_features, distance_constraints,
           coordination_constraints, params):
    p = params
    npad = _EPAD - E
    gpad = (jnp.arange(npad, dtype=jnp.int32) * 37) % N
    rowg = jnp.concatenate([edge_index[0], gpad])
    colg = jnp.concatenate([edge_index[1], gpad])
    spad = N + (jnp.arange(npad, dtype=jnp.int32) % (_NP - N))
    rows = jnp.concatenate([edge_index[0], spad])
    zm = jnp.zeros((_NP // _NS, H), jnp.float32)
    zx = jnp.zeros((_NP * 4 // _NS,), jnp.float32)
    eye8 = jnp.eye(8, dtype=jnp.float32)
    msk8 = jnp.array([[1.], [1.], [1.], [0.]], jnp.float32)

    h = h + _precompute(t, anchor_features, distance_constraints,
                        coordination_constraints, p)
    hrow = _EROWS // 2
    for i in range(2):
        x4 = jnp.pad(x, ((0, 0), (0, 1))).reshape(-1)
        bf = jnp.bfloat16
        wts = (p['eW1'][i, :2 * H].astype(bf),
               p['eW1'][i, 2 * H:].astype(bf), p['eb1'][i][None].astype(bf),
               p['eW2'][i].astype(bf), p['eb2'][i][None].astype(bf),
               p['aW'][i].astype(bf), p['ab'][i][None],
               p['cW1'][i].astype(bf), p['cb1'][i][None].astype(bf),
               p['cW2'][i].T.astype(bf))
        parts = []
        for hs in (slice(0, hrow * _C), slice(hrow * _C, _EPAD)):
            hr, hc, rel8 = _sc_gather(h, x4, rowg[hs], colg[hs])
            msg, xmsg = _edge_block_call(hr, hc, rel8, eye8, msk8, wts)
            parts.append(_sc_scatter(msg, xmsg, rows[hs], zm, zx))
        (mo1, xo1), (mo2, xo2) = parts
        h = _node_call(i == 1, h, mo1, mo2,
                       p['nW1'][i, :H], p['nW1'][i, H:], p['nb1'][i][None],
                       p['nW2'][i], p['nb2'][i][None],
                       p['ln_g'][None], p['ln_b'][None])
        xr = (xo1[0] + xo1[1] + xo2[0] + xo2[1]).reshape(_NP, 4)
        x = x + xr[:N, :3]
    return h, x


# speedup vs baseline: 5.9130x; 1.0012x over previous
import dataclasses
import functools
import math

import jax
import jax.numpy as jnp
from jax import lax
from jax.experimental import pallas as pl
from jax.experimental.pallas import tpu as pltpu
from jax.experimental.pallas import tpu_sc as plsc

N = 10000
E = 160000
H = 128

_NC = 2
_NS = 16
_NW = _NC * _NS
_C = 128
_EROWS = 1280
_EPAD = _EROWS * _C
_RPW = _EROWS // _NW
_NP = 10240

_BE = 1024
_BN = 1000

_sc_mesh = plsc.VectorSubcoreMesh(core_axis_name="c", subcore_axis_name="s")

_sc_cp = pltpu.CompilerParams()
if "needs_layout_passes" in pltpu.CompilerParams.__dataclass_fields__:
    _sc_cp = dataclasses.replace(_sc_cp, needs_layout_passes=False)


def _sc_gather(h2b, x4, rowg, colg):
    nrows = rowg.shape[0] // _C
    rpw = nrows // _NW
    npad = nrows * _C

    @functools.partial(
        pl.kernel, mesh=_sc_mesh,
        out_type=[jax.ShapeDtypeStruct((npad, H), jnp.float32),
                  jax.ShapeDtypeStruct((npad, H), jnp.float32),
                  jax.ShapeDtypeStruct((8, npad), jnp.float32)],
        scratch_types=[pltpu.VMEM((rpw, _C), jnp.int32),
                       pltpu.VMEM((rpw, _C), jnp.int32),
                       pltpu.VMEM((_C, H), jnp.float32),
                       pltpu.VMEM((_C, H), jnp.float32),
                       pltpu.VMEM((_C, H), jnp.float32),
                       pltpu.VMEM((_C, H), jnp.float32),
                       pltpu.VMEM((4 * N,), jnp.float32),
                       pltpu.VMEM((8, _C), jnp.float32),
                       pltpu.VMEM((8, _C), jnp.float32),
                       pltpu.SemaphoreType.DMA,
                       pltpu.SemaphoreType.DMA],
        compiler_params=_sc_cp,
    )
    def k(h_hbm, x_hbm, ri_hbm, ci_hbm, hr_hbm, hc_hbm, rel_hbm,
          ribuf, cibuf, hbuf0, cbuf0, hbuf1, cbuf1, x4v, relbuf0, relbuf1,
          semg, semw):
        wid = lax.axis_index("s") * _NC + lax.axis_index("c")
        base = wid * rpw
        idma = []
        for j in range(rpw):
            idma.append(pltpu.async_copy(
                ri_hbm.at[pl.ds((base + j) * _C, _C)], ribuf.at[j], semg))
            idma.append(pltpu.async_copy(
                ci_hbm.at[pl.ds((base + j) * _C, _C)], cibuf.at[j], semg))
        pltpu.sync_copy(x_hbm, x4v)
        for d in idma:
            d.wait()
        zero16 = jnp.zeros((16,), jnp.float32)
        for rb in (relbuf0, relbuf1):
            for r in range(4, 8):
                for kk in range(8):
                    rb[r, pl.ds(kk * 16, 16)] = zero16

        def relcompute(j, rb):
            for kk in range(8):
                sl = pl.ds(kk * 16, 16)
                ir = ribuf[j, sl] * 4
                ic = cibuf[j, sl] * 4
                d2 = zero16
                for comp in range(3):
                    cidx = jnp.full((16,), comp, jnp.int32)
                    vr = plsc.load_gather(x4v, [ir + cidx])
                    vc = plsc.load_gather(x4v, [ic + cidx])
                    rr = vr - vc
                    rb[comp, sl] = rr
                    d2 = d2 + rr * rr
                rb[3, sl] = d2

        def drain_writes():
            pltpu.make_async_copy(hr_hbm.at[pl.ds(0, _C)], hbuf0,
                                  semw).wait()
            pltpu.make_async_copy(hc_hbm.at[pl.ds(0, _C)], cbuf0,
                                  semw).wait()
            pltpu.make_async_copy(rel_hbm.at[:, pl.ds(0, _C)], relbuf0,
                                  semw).wait()

        @pl.loop(0, rpw // 2)
        def _(jp):
            j0 = jp * 2

            @pl.when(jp > 0)
            def _():
                drain_writes()
                drain_writes()

            g0a = pltpu.async_copy(h_hbm.at[ribuf.at[j0]], hbuf0, semg)
            g0b = pltpu.async_copy(h_hbm.at[cibuf.at[j0]], cbuf0, semg)
            g1a = pltpu.async_copy(h_hbm.at[ribuf.at[j0 + 1]], hbuf1, semg)
            g1b = pltpu.async_copy(h_hbm.at[cibuf.at[j0 + 1]], cbuf1, semg)
            relcompute(j0, relbuf0)
            e0 = (base + j0) * _C
            g0a.wait()
            g0b.wait()
            pltpu.async_copy(hbuf0, hr_hbm.at[pl.ds(e0, _C)], semw)
            pltpu.async_copy(cbuf0, hc_hbm.at[pl.ds(e0, _C)], semw)
            pltpu.async_copy(relbuf0, rel_hbm.at[:, pl.ds(e0, _C)], semw)
            relcompute(j0 + 1, relbuf1)
            g1a.wait()
            g1b.wait()
            pltpu.async_copy(hbuf1, hr_hbm.at[pl.ds(e0 + _C, _C)], semw)
            pltpu.async_copy(cbuf1, hc_hbm.at[pl.ds(e0 + _C, _C)], semw)
            pltpu.async_copy(relbuf1, rel_hbm.at[:, pl.ds(e0 + _C, _C)],
                             semw)

        drain_writes()
        drain_writes()

    return k(h2b, x4, rowg, colg)


def _sc_scatter(msg, xmsg, rows, zm, zx):
    rps = _NP // _NS
    rpc = rows.shape[0] // _C // _NC
    rps_e = rpc // _NS

    nx = _NP * 4
    nxs = nx // _NS

    @functools.partial(
        pl.kernel, mesh=_sc_mesh,
        out_type=[jax.ShapeDtypeStruct((_NC, _NP, H), jnp.float32),
                  jax.ShapeDtypeStruct((_NC, nx), jnp.float32)],
        scratch_types=[pltpu.VMEM((rps_e, _C), jnp.int32),
                       pltpu.VMEM((_C, H), jnp.float32),
                       pltpu.VMEM((_C, H), jnp.float32),
                       pltpu.VMEM((4, _C), jnp.float32),
                       pltpu.VMEM((4, _C), jnp.float32),
                       pltpu.VMEM((4, _C), jnp.int32),
                       pltpu.VMEM_SHARED((_NP, H), jnp.float32),
                       pltpu.VMEM_SHARED((nx,), jnp.float32),
                       pltpu.SemaphoreType.DMA],
        compiler_params=_sc_cp,
    )
    def k(m_hbm, xm_hbm, ri_hbm, zm_hbm, zx_hbm, mo_hbm, xo_hbm,
          ibuf, mbuf0, mbuf1, xbuf0, xbuf1, ixbuf, macc, xacc, sem):
        c = lax.axis_index("c")
        s = lax.axis_index("s")
        pltpu.sync_copy(zm_hbm, macc.at[pl.ds(s * rps, rps)])
        pltpu.sync_copy(zx_hbm, xacc.at[pl.ds(s * nxs, nxs)])
        plsc.subcore_barrier()
        base = c * rpc + s * rps_e
        idma = []
        for j in range(rps_e):
            idma.append(pltpu.async_copy(
                ri_hbm.at[pl.ds((base + j) * _C, _C)], ibuf.at[j], sem))
        for d in idma:
            d.wait()

        def xscat(j, xbuf):
            for r in range(3):
                for g in range(8):
                    sl = pl.ds(g * 16, 16)
                    ixbuf[r, sl] = ibuf[j, sl] * 4 + r
            for r in range(3):
                pltpu.sync_copy(xbuf.at[r], xacc.at[ixbuf.at[r]],
                                add=True)

        @pl.loop(0, rps_e // 2)
        def _(jp):
            j0 = jp * 2
            e0 = (base + j0) * _C
            d0 = pltpu.async_copy(m_hbm.at[pl.ds(e0, _C)], mbuf0, sem)
            dx0 = pltpu.async_copy(xm_hbm.at[:, pl.ds(e0, _C)], xbuf0, sem)
            d1 = pltpu.async_copy(m_hbm.at[pl.ds(e0 + _C, _C)], mbuf1, sem)
            dx1 = pltpu.async_copy(xm_hbm.at[:, pl.ds(e0 + _C, _C)], xbuf1,
                                   sem)
            d0.wait()
            dx0.wait()
            pltpu.sync_copy(mbuf0, macc.at[ibuf.at[j0]], add=True)
            xscat(j0, xbuf0)
            d1.wait()
            dx1.wait()
            pltpu.sync_copy(mbuf1, macc.at[ibuf.at[j0 + 1]], add=True)
            xscat(j0 + 1, xbuf1)

        plsc.subcore_barrier()
        pltpu.sync_copy(macc.at[pl.ds(s * rps, rps)],
                        mo_hbm.at[c, pl.ds(s * rps, rps)])
        pltpu.sync_copy(xacc.at[pl.ds(s * nxs, nxs)],
                        xo_hbm.at[c, pl.ds(s * nxs, nxs)])

    return k(msg, xmsg, rows, zm, zx)


def _silu(v):
    return v * jax.nn.sigmoid(v)


def _edge_body(hr, hc, rel8, eye8, msk8, w1cat, w1d, b1, w2, b2,
               aw, ab, cw1, cb1, cw2, msg, xout):
    bf = jnp.bfloat16
    f32 = jnp.float32
    r8 = lax.dot_general(rel8[...], eye8[...], (((0,), (0,)), ((), ())),
                         preferred_element_type=f32)
    distb = jnp.sqrt(r8[:, 3:4]).astype(bf)
    hh = jnp.concatenate([hr[...].astype(bf), hc[...].astype(bf)], axis=1)
    t1 = (jnp.dot(hh, w1cat[...], preferred_element_type=f32).astype(bf)
          + distb * w1d[...] + b1[...])
    t1 = _silu(t1)
    m = jnp.dot(t1, w2[...], preferred_element_type=f32).astype(bf) + b2[...]
    m = _silu(m)
    att = jax.nn.sigmoid(jnp.dot(m, aw[...], preferred_element_type=f32)
                         + ab[...])
    msg[...] = att * m.astype(f32)
    c1 = _silu(jnp.dot(m, cw1[...], preferred_element_type=f32).astype(bf)
               + cb1[...])
    cwT = lax.dot_general(cw2[...], c1, (((1,), (1,)), ((), ())),
                          preferred_element_type=f32)
    distT = jnp.sqrt(rel8[3:4, :])
    xout[...] = (cwT * rel8[0:4, :] / (distT + 1e-8)) * msk8[...]


def _edge_block_call(hr, hc, rel8, eye8, msk8, wts):
    n_e = hr.shape[0]
    grid = n_e // _BE
    full = lambda s: pl.BlockSpec(s, lambda i: (0,) * len(s))
    eb = lambda d: pl.BlockSpec((_BE, d), lambda i: (i, 0))
    return pl.pallas_call(
        _edge_body,
        grid=(grid,),
        in_specs=[eb(H), eb(H), pl.BlockSpec((8, _BE), lambda i: (0, i)),
                  full((8, 8)), full((4, 1)),
                  full((2 * H, H)), full((1, H)), full((1, H)),
                  full((H, H)), full((1, H)), full((H, 1)), full((1, 1)),
                  full((H, H)), full((1, H)), full((1, H))],
        out_specs=[eb(H), pl.BlockSpec((4, _BE), lambda i: (0, i))],
        out_shape=[jax.ShapeDtypeStruct((n_e, H), jnp.float32),
                   jax.ShapeDtypeStruct((4, n_e), jnp.float32)],
        compiler_params=pltpu.CompilerParams(
            dimension_semantics=("parallel",)),
    )(hr, hc, rel8, eye8, msk8, *wts)


def _node_body(ln, nmo, h, *refs):
    ms = refs[:2 * nmo]
    w1a, w1b, b1, w2, b2, g, bv, out = refs[2 * nmo:]
    mi = ms[0][0]
    for mr in ms[1:]:
        mi = mi + mr[0]
    t = (jnp.dot(h[...], w1a[...], preferred_element_type=jnp.float32)
         + jnp.dot(mi, w1b[...], preferred_element_type=jnp.float32)
         + b1[...])
    t = _silu(t)
    hn = jnp.dot(t, w2[...], preferred_element_type=jnp.float32) + b2[...]
    hnew = h[...] + hn
    if ln:
        mu = jnp.mean(hnew, axis=-1, keepdims=True)
        va = jnp.mean((hnew - mu) ** 2, axis=-1, keepdims=True)
        hnew = (hnew - mu) / jnp.sqrt(va + 1e-5) * g[...] + bv[...]
    out[...] = hnew


def _node_call(ln, h, mos, w1a, w1b, b1, w2, b2, g, bv):
    grid = N // _BN
    full = lambda s: pl.BlockSpec(s, lambda i: (0,) * len(s))
    nb = pl.BlockSpec((_BN, H), lambda i: (i, 0))
    m0 = pl.BlockSpec((1, _BN, H), lambda i: (0, i, 0))
    m1 = pl.BlockSpec((1, _BN, H), lambda i: (1, i, 0))
    mspecs = []
    margs = []
    for mo in mos:
        mspecs += [m0, m1]
        margs += [mo, mo]
    return pl.pallas_call(
        functools.partial(_node_body, ln, len(mos)),
        grid=(grid,),
        in_specs=[nb] + mspecs +
                 [full((H, H)), full((H, H)), full((1, H)),
                  full((H, H)), full((1, H)), full((1, H)), full((1, H))],
        out_specs=nb,
        out_shape=jax.ShapeDtypeStruct((N, H), jnp.float32),
        compiler_params=pltpu.CompilerParams(
            dimension_semantics=("parallel",)),
    )(h, *margs, w1a, w1b, b1, w2, b2, g, bv)


def _precompute(t, af, dc, cc, p):
    silu = jax.nn.silu
    half = H // 2
    freqs = jnp.exp(jnp.arange(half, dtype=jnp.float32)
                    * (-(math.log(10000.0) / (half - 1))))
    te = t.astype(jnp.float32)[:, None] * freqs[None, :]
    temb = jnp.concatenate([jnp.sin(te), jnp.cos(te)], axis=-1)
    a = silu(af @ p['ce_aW1'] + p['ce_ab1']) @ p['ce_aW2'] + p['ce_ab2']
    a_emb = a.mean(axis=0, keepdims=True)
    d = silu(dc @ p['ce_dW1'] + p['ce_db1']) @ p['ce_dW2'] + p['ce_db2']
    d_emb = d.mean(axis=0, keepdims=True)
    c = silu(cc @ p['ce_cW1'] + p['ce_cb1']) @ p['ce_cW2'] + p['ce_cb2']
    c_emb = c.mean(axis=0, keepdims=True)
    comb = jnp.concatenate([a_emb, d_emb, c_emb], axis=-1)
    z = comb @ p['ce_fW1'] + p['ce_fb1']
    mu = z.mean(axis=-1, keepdims=True)
    va = ((z - mu) ** 2).mean(axis=-1, keepdims=True)
    z = (z - mu) / jnp.sqrt(va + 1e-5) * p['ce_fg'] + p['ce_fbe']
    cond = silu(z) @ p['ce_fW2'] + p['ce_fb2']
    tproj = temb @ p['tpW'] + p['tpb']
    cproj = cond @ p['cpW'] + p['cpb']
    return tproj + cproj


def kernel(h, x, edge_index, t, anchor_features, distance_constraints,
           coordination_constraints, params):
    p = params
    npad = _EPAD - E
    gpad = (jnp.arange(npad, dtype=jnp.int32) * 37) % N
    rowg = jnp.concatenate([edge_index[0], gpad])
    colg = jnp.concatenate([edge_index[1], gpad])
    spad = N + (jnp.arange(npad, dtype=jnp.int32) % (_NP - N))
    rows = jnp.concatenate([edge_index[0], spad])
    zm = jnp.zeros((_NP // _NS, H), jnp.float32)
    zx = jnp.zeros((_NP * 4 // _NS,), jnp.float32)
    eye8 = jnp.eye(8, dtype=jnp.float32)
    msk8 = jnp.array([[1.], [1.], [1.], [0.]], jnp.float32)

    h = h + _precompute(t, anchor_features, distance_constraints,
                        coordination_constraints, p)
    nchunk = 4
    crow = _EPAD // nchunk
    for i in range(2):
        x4 = jnp.pad(x, ((0, 0), (0, 1))).reshape(-1)
        bf = jnp.bfloat16
        wts = (p['eW1'][i, :2 * H].astype(bf),
               p['eW1'][i, 2 * H:].astype(bf), p['eb1'][i][None].astype(bf),
               p['eW2'][i].astype(bf), p['eb2'][i][None].astype(bf),
               p['aW'][i].astype(bf), p['ab'][i][None],
               p['cW1'][i].astype(bf), p['cb1'][i][None].astype(bf),
               p['cW2'][i].T.astype(bf))
        parts = []
        for ci in range(nchunk):
            hs = slice(ci * crow, (ci + 1) * crow)
            hr, hc, rel8 = _sc_gather(h, x4, rowg[hs], colg[hs])
            msg, xmsg = _edge_block_call(hr, hc, rel8, eye8, msk8, wts)
            parts.append(_sc_scatter(msg, xmsg, rows[hs], zm, zx))
        h = _node_call(i == 1, h, [mo for mo, _ in parts],
                       p['nW1'][i, :H], p['nW1'][i, H:], p['nb1'][i][None],
                       p['nW2'][i], p['nb2'][i][None],
                       p['ln_g'][None], p['ln_b'][None])
        xacc = parts[0][1][0] + parts[0][1][1]
        for _, xo in parts[1:]:
            xacc = xacc + xo[0] + xo[1]
        x = x + xacc.reshape(_NP, 4)[:N, :3]
    return h, x
